# baseline (speedup 1.0000x reference)
import functools

import jax
import jax.numpy as jnp
from jax import lax
from jax.experimental import pallas as pl
from jax.experimental.pallas import tpu as pltpu
from jax.experimental.pallas import tpu_sc as plsc

USER_N = 5000
N = 10000
E = 320000
D = 128
L = 16
NC = 2
NS = 16
NW = NC * NS
C = 96
CHUNKS = 112
EPW = CHUNKS * C
E_PAD = NW * EPW
ROWS_PER_SUB = 624

_GATHER_DN = lax.GatherDimensionNumbers(
    offset_dims=(), collapsed_slice_dims=(0,), start_index_map=(0,))


def _splat(v16, t):
    idx = jnp.full((L, 1), t, jnp.int32)
    return lax.gather(v16, idx, _GATHER_DN, (1,),
                      mode=lax.GatherScatterMode.PROMISE_IN_BOUNDS)


def _sc_layer_body(ego_hbm, col_hbm, row_hbm, val_hbm, out_hbm,
                   ridx_all, vals_all, cidx_a, cidx_b, rows_a, rows_b, acc,
                   gsem_a, gsem_b, isem_a, isem_b, ssem_a, ssem_b):
    c = lax.axis_index("c")
    s = lax.axis_index("s")
    w = s * NC + c

    pltpu.sync_copy(row_hbm.at[pl.ds(w * CHUNKS, CHUNKS)], ridx_all)
    pltpu.sync_copy(val_hbm.at[pl.ds(w * EPW, EPW)], vals_all)

    base0 = w * EPW
    pltpu.sync_copy(col_hbm.at[pl.ds(base0, C)], cidx_a)
    pltpu.async_copy(ego_hbm.at[cidx_a], rows_a, gsem_a)
    pltpu.async_copy(col_hbm.at[pl.ds(base0 + C, C)], cidx_b, isem_b)

    zero = jnp.zeros((L,), jnp.float32)

    def _zero_rows(i, carry):
        for j in range(D // L):
            rows_b[i, pl.ds(j * L, L)] = zero
        return carry

    lax.fori_loop(0, C, _zero_rows, 0)
    for b in range(ROWS_PER_SUB // C):
        pltpu.sync_copy(rows_b.at[pl.ds(0, C)],
                        acc.at[pl.ds(s * ROWS_PER_SUB + b * C, C)])
    rem = ROWS_PER_SUB % C
    pltpu.sync_copy(rows_b.at[pl.ds(0, rem)],
                    acc.at[pl.ds(s * ROWS_PER_SUB + ROWS_PER_SUB - rem, rem)])

    @pl.when(s == NS - 1)
    def _zero_tail():
        pltpu.sync_copy(rows_b.at[pl.ds(0, N - NS * ROWS_PER_SUB)],
                        acc.at[pl.ds(NS * ROWS_PER_SUB, N - NS * ROWS_PER_SUB)])

    plsc.subcore_barrier()

    def _process(k, cidx_x, rows_x, gsem_x, isem_x, ssem_x,
                 cidx_y, rows_y, gsem_y, isem_y, ssem_y):
        pltpu.make_async_copy(ego_hbm.at[cidx_x], rows_x, gsem_x).wait()

        @pl.when(k + 2 < CHUNKS)
        def _prefetch_idx():
            pltpu.async_copy(col_hbm.at[pl.ds(base0 + (k + 2) * C, C)],
                             cidx_x, isem_x)

        @pl.when(k + 1 < CHUNKS)
        def _launch_gather():
            @pl.when(k >= 1)
            def _drain_scatter():
                pltpu.make_async_copy(rows_y, acc.at[ridx_all.at[k]],
                                      ssem_y).wait()

            pltpu.make_async_copy(col_hbm.at[pl.ds(base0 + (k + 1) * C, C)],
                                  cidx_y, isem_y).wait()
            pltpu.async_copy(ego_hbm.at[cidx_y], rows_y, gsem_y)

        def _scale16(g, inner):
            v16 = vals_all[pl.ds(k * C + g * L, L)]
            for t in range(L):
                sp = _splat(v16, t)
                e = g * L + t
                for j in range(D // L):
                    rows_x[e, pl.ds(j * L, L)] = rows_x[e, pl.ds(j * L, L)] * sp
            return inner

        lax.fori_loop(0, C // L, _scale16, 0)
        pltpu.async_copy(rows_x, acc.at[ridx_all.at[k]], ssem_x, add=True)

    def _chunk(k, carry):
        @pl.when(k % 2 == 0)
        def _even():
            _process(k, cidx_a, rows_a, gsem_a, isem_a, ssem_a,
                     cidx_b, rows_b, gsem_b, isem_b, ssem_b)

        @pl.when(k % 2 == 1)
        def _odd():
            _process(k, cidx_b, rows_b, gsem_b, isem_b, ssem_b,
                     cidx_a, rows_a, gsem_a, isem_a, ssem_a)

        return carry

    lax.fori_loop(0, CHUNKS, _chunk, 0)
    pltpu.make_async_copy(rows_a, acc.at[ridx_all.at[CHUNKS - 2]],
                          ssem_a).wait()
    pltpu.make_async_copy(rows_b, acc.at[ridx_all.at[CHUNKS - 1]],
                          ssem_b).wait()
    plsc.subcore_barrier()

    pltpu.sync_copy(acc.at[pl.ds(s * ROWS_PER_SUB, ROWS_PER_SUB)],
                    out_hbm.at[pl.ds(c * N + s * ROWS_PER_SUB, ROWS_PER_SUB)])

    @pl.when(s == NS - 1)
    def _write_tail():
        pltpu.sync_copy(acc.at[pl.ds(NS * ROWS_PER_SUB, N - NS * ROWS_PER_SUB)],
                        out_hbm.at[pl.ds(c * N + NS * ROWS_PER_SUB,
                                         N - NS * ROWS_PER_SUB)])


_sc_layer = functools.partial(
    pl.kernel,
    out_type=jax.ShapeDtypeStruct((NC * N, D), jnp.float32),
    mesh=plsc.VectorSubcoreMesh(core_axis_name="c", subcore_axis_name="s"),
    scratch_types=[
        pltpu.VMEM((CHUNKS, C), jnp.int32),
        pltpu.VMEM((EPW,), jnp.float32),
        pltpu.VMEM((C,), jnp.int32),
        pltpu.VMEM((C,), jnp.int32),
        pltpu.VMEM((C, D), jnp.float32),
        pltpu.VMEM((C, D), jnp.float32),
        pltpu.VMEM_SHARED((N, D), jnp.float32),
        pltpu.SemaphoreType.DMA,
        pltpu.SemaphoreType.DMA,
        pltpu.SemaphoreType.DMA,
        pltpu.SemaphoreType.DMA,
        pltpu.SemaphoreType.DMA,
        pltpu.SemaphoreType.DMA,
    ],
)(_sc_layer_body)


def _add_halves_body(a_ref, b_ref, o_ref):
    o_ref[...] = a_ref[...] + b_ref[...]


def _add_halves(p):
    blk = N // 10
    return pl.pallas_call(
        _add_halves_body,
        out_shape=jax.ShapeDtypeStruct((N, D), jnp.float32),
        grid=(10,),
        in_specs=[pl.BlockSpec((blk, D), lambda i: (i, 0)),
                  pl.BlockSpec((blk, D), lambda i: (i + 10, 0))],
        out_specs=pl.BlockSpec((blk, D), lambda i: (i, 0)),
    )(p, p)


def _mean_body(p3a_ref, p3b_ref, e1_ref, e2_ref, o_ref):
    o_ref[...] = (p3a_ref[...] + p3b_ref[...] + e1_ref[...] + e2_ref[...]) * (
        jnp.float32(1.0 / 3.0))


def _user_mean(p3, e1, e2):
    blk = USER_N // 5
    return pl.pallas_call(
        _mean_body,
        out_shape=jax.ShapeDtypeStruct((USER_N, D), jnp.float32),
        grid=(5,),
        in_specs=[
            pl.BlockSpec((blk, D), lambda i: (i, 0)),
            pl.BlockSpec((blk, D), lambda i: (i + N // blk, 0)),
            pl.BlockSpec((blk, D), lambda i: (i, 0)),
            pl.BlockSpec((blk, D), lambda i: (i, 0)),
        ],
        out_specs=pl.BlockSpec((blk, D), lambda i: (i, 0)),
    )(p3, p3, e1, e2)


def kernel(user_emb, item_emb, adj_values, adj_indices):
    ego0 = jnp.concatenate([user_emb, item_emb], axis=0)
    row = adj_indices[0].astype(jnp.int32)
    col = adj_indices[1].astype(jnp.int32)
    pad = E_PAD - E
    zpad_i = jnp.zeros((pad,), jnp.int32)
    colp = jnp.concatenate([col, zpad_i])
    rowp = jnp.concatenate([row, zpad_i]).reshape(NW * CHUNKS, C)
    valp = jnp.concatenate([adj_values, jnp.zeros((pad,), jnp.float32)])

    p1 = _sc_layer(ego0, colp, rowp, valp)
    e1 = _add_halves(p1)
    p2 = _sc_layer(e1, colp, rowp, valp)
    e2 = _add_halves(p2)
    p3 = _sc_layer(e2, colp, rowp, valp)
    user_out = _user_mean(p3, e1, e2)
    return (user_out, item_emb)


# speedup vs baseline: 1.7184x; 1.7184x over previous
import functools

import numpy as np
import jax
import jax.numpy as jnp
from jax import lax
from jax.experimental import pallas as pl
from jax.experimental.pallas import tpu as pltpu
from jax.experimental.pallas import tpu_sc as plsc

USER_N = 5000
N = 10000
E = 320000
D = 128
L = 16
NC = 2
NS = 16
NW = NC * NS
C = 96
CHUNKS = 112
EPW = CHUNKS * C
E_PAD = NW * EPW
ROWS_PER_SUB = 624

_HI_MASK = np.int32(-65536)

_PERM = np.empty((D,), np.int32)
for _m in range(D // 32):
    _b = 32 * _m
    for _i in range(16):
        _PERM[_b + _i] = _b + 2 * _i
        _PERM[_b + 16 + _i] = _b + 2 * _i + 1
_UNPERM_MAT = np.zeros((D, D), np.float32)
_UNPERM_MAT[np.arange(D), _PERM] = 1.0

_GATHER_DN = lax.GatherDimensionNumbers(
    offset_dims=(), collapsed_slice_dims=(0,), start_index_map=(0,))


def _splat(v16, t):
    idx = jnp.full((L, 1), t, jnp.int32)
    return lax.gather(v16, idx, _GATHER_DN, (1,),
                      mode=lax.GatherScatterMode.PROMISE_IN_BOUNDS)


def _sc_layer_body(ego_hbm, col_hbm, row_hbm, val_hbm, out_hbm,
                   cidx_all, vals_all, ridx_a, ridx_b, rows_a, rows_b, msgs,
                   acc, gsem_a, gsem_b, rsem_a, rsem_b):
    c = lax.axis_index("c")
    s = lax.axis_index("s")
    w = s * NC + c

    pltpu.sync_copy(col_hbm.at[pl.ds(w * EPW, EPW)], cidx_all)
    pltpu.sync_copy(val_hbm.at[pl.ds(w * EPW, EPW)], vals_all)

    zero = jnp.zeros((L,), jnp.float32)

    def _zero_rows(i, carry):
        for j in range(D // L):
            msgs[i, pl.ds(j * L, L)] = zero
        return carry

    lax.fori_loop(0, C, _zero_rows, 0)
    for b in range(ROWS_PER_SUB // C):
        pltpu.sync_copy(msgs.at[pl.ds(0, C)],
                        acc.at[pl.ds(s * ROWS_PER_SUB + b * C, C)])
    rem = ROWS_PER_SUB % C
    pltpu.sync_copy(msgs.at[pl.ds(0, rem)],
                    acc.at[pl.ds(s * ROWS_PER_SUB + ROWS_PER_SUB - rem, rem)])

    @pl.when(s == NS - 1)
    def _zero_tail():
        pltpu.sync_copy(msgs.at[pl.ds(0, N - NS * ROWS_PER_SUB)],
                        acc.at[pl.ds(NS * ROWS_PER_SUB, N - NS * ROWS_PER_SUB)])

    plsc.subcore_barrier()

    base0 = w * EPW
    pltpu.async_copy(row_hbm.at[pl.ds(base0, C)], ridx_a, rsem_a)
    pltpu.async_copy(ego_hbm.at[cidx_all.at[pl.ds(0, C)]], rows_a, gsem_a)

    def _process(k, rows_x, ridx_x, gsem_x, rsem_x,
                 rows_y, ridx_y, gsem_y, rsem_y):
        @pl.when(k + 1 < CHUNKS)
        def _prefetch():
            pltpu.async_copy(row_hbm.at[pl.ds(base0 + (k + 1) * C, C)],
                             ridx_y, rsem_y)
            pltpu.async_copy(ego_hbm.at[cidx_all.at[pl.ds((k + 1) * C, C)]],
                             rows_y, gsem_y)

        pltpu.make_async_copy(ego_hbm.at[cidx_all.at[pl.ds(k * C, C)]],
                              rows_x, gsem_x).wait()

        def _scale16(g, inner):
            v16 = vals_all[pl.ds(k * C + g * L, L)]
            for t in range(L):
                sp = _splat(v16, t)
                e = g * L + t
                for m in range(D // 32):
                    u = rows_x[e, pl.ds(m * L, L)]
                    even = lax.bitcast_convert_type(u << 16, jnp.float32)
                    odd = lax.bitcast_convert_type(u & _HI_MASK, jnp.float32)
                    msgs[e, pl.ds(m * 32, L)] = even * sp
                    msgs[e, pl.ds(m * 32 + L, L)] = odd * sp
            return inner

        lax.fori_loop(0, C // L, _scale16, 0)
        pltpu.make_async_copy(row_hbm.at[pl.ds(base0 + k * C, C)],
                              ridx_x, rsem_x).wait()
        pltpu.sync_copy(msgs, acc.at[ridx_x], add=True)

    def _chunk(k, carry):
        @pl.when(k % 2 == 0)
        def _even():
            _process(k, rows_a, ridx_a, gsem_a, rsem_a,
                     rows_b, ridx_b, gsem_b, rsem_b)

        @pl.when(k % 2 == 1)
        def _odd():
            _process(k, rows_b, ridx_b, gsem_b, rsem_b,
                     rows_a, ridx_a, gsem_a, rsem_a)

        return carry

    lax.fori_loop(0, CHUNKS, _chunk, 0)
    plsc.subcore_barrier()

    pltpu.sync_copy(acc.at[pl.ds(s * ROWS_PER_SUB, ROWS_PER_SUB)],
                    out_hbm.at[pl.ds(c * N + s * ROWS_PER_SUB, ROWS_PER_SUB)])

    @pl.when(s == NS - 1)
    def _write_tail():
        pltpu.sync_copy(acc.at[pl.ds(NS * ROWS_PER_SUB, N - NS * ROWS_PER_SUB)],
                        out_hbm.at[pl.ds(c * N + NS * ROWS_PER_SUB,
                                         N - NS * ROWS_PER_SUB)])


_sc_layer = functools.partial(
    pl.kernel,
    out_type=jax.ShapeDtypeStruct((NC * N, D), jnp.float32),
    mesh=plsc.VectorSubcoreMesh(core_axis_name="c", subcore_axis_name="s"),
    compiler_params=pltpu.CompilerParams(
        needs_layout_passes=False, use_tc_tiling_on_sc=False),
    scratch_types=[
        pltpu.VMEM((EPW,), jnp.int32),
        pltpu.VMEM((EPW,), jnp.float32),
        pltpu.VMEM((C,), jnp.int32),
        pltpu.VMEM((C,), jnp.int32),
        pltpu.VMEM((C, D // 2), jnp.int32),
        pltpu.VMEM((C, D // 2), jnp.int32),
        pltpu.VMEM((C, D), jnp.float32),
        pltpu.VMEM_SHARED((N, D), jnp.float32),
        pltpu.SemaphoreType.DMA,
        pltpu.SemaphoreType.DMA,
        pltpu.SemaphoreType.DMA,
        pltpu.SemaphoreType.DMA,
    ],
)(_sc_layer_body)


def _combine_body(a_ref, b_ref, m_ref, operm_ref, obf_ref):
    x = a_ref[...] + b_ref[...]
    operm_ref[...] = x
    obf_ref[...] = jnp.dot(
        x, m_ref[...], preferred_element_type=jnp.float32).astype(jnp.bfloat16)


def _combine(p, unperm):
    blk = N // 10
    return pl.pallas_call(
        _combine_body,
        out_shape=(jax.ShapeDtypeStruct((N, D), jnp.float32),
                   jax.ShapeDtypeStruct((N, D), jnp.bfloat16)),
        grid=(10,),
        in_specs=[pl.BlockSpec((blk, D), lambda i: (i, 0)),
                  pl.BlockSpec((blk, D), lambda i: (i + 10, 0)),
                  pl.BlockSpec((D, D), lambda i: (0, 0))],
        out_specs=(pl.BlockSpec((blk, D), lambda i: (i, 0)),
                   pl.BlockSpec((blk, D), lambda i: (i, 0))),
    )(p, p, unperm)


def _mean_body(p3a_ref, p3b_ref, e1_ref, e2_ref, m_ref, o_ref):
    x = p3a_ref[...] + p3b_ref[...] + e1_ref[...] + e2_ref[...]
    o_ref[...] = jnp.dot(
        x, m_ref[...], preferred_element_type=jnp.float32) * (
            jnp.float32(1.0 / 3.0))


def _user_mean(p3, e1, e2, unperm):
    blk = USER_N // 5
    return pl.pallas_call(
        _mean_body,
        out_shape=jax.ShapeDtypeStruct((USER_N, D), jnp.float32),
        grid=(5,),
        in_specs=[
            pl.BlockSpec((blk, D), lambda i: (i, 0)),
            pl.BlockSpec((blk, D), lambda i: (i + N // blk, 0)),
            pl.BlockSpec((blk, D), lambda i: (i, 0)),
            pl.BlockSpec((blk, D), lambda i: (i, 0)),
            pl.BlockSpec((D, D), lambda i: (0, 0)),
        ],
        out_specs=pl.BlockSpec((blk, D), lambda i: (i, 0)),
    )(p3, p3, e1, e2, unperm)


def _pack_table(x_bf16):
    return lax.bitcast_convert_type(
        x_bf16.reshape(N, D // 2, 2), jnp.int32)


def kernel(user_emb, item_emb, adj_values, adj_indices):
    ego0 = jnp.concatenate([user_emb, item_emb], axis=0)
    row = adj_indices[0].astype(jnp.int32)
    col = adj_indices[1].astype(jnp.int32)
    pad = E_PAD - E
    zpad_i = jnp.zeros((pad,), jnp.int32)
    colp = jnp.concatenate([col, zpad_i])
    rowp = jnp.concatenate([row, zpad_i])
    valp = jnp.concatenate([adj_values, jnp.zeros((pad,), jnp.float32)])
    unperm = jnp.asarray(_UNPERM_MAT)

    p1 = _sc_layer(_pack_table(ego0.astype(jnp.bfloat16)), colp, rowp, valp)
    e1, e1_bf = _combine(p1, unperm)
    p2 = _sc_layer(_pack_table(e1_bf), colp, rowp, valp)
    e2, e2_bf = _combine(p2, unperm)
    p3 = _sc_layer(_pack_table(e2_bf), colp, rowp, valp)
    user_out = _user_mean(p3, e1, e2, unperm)
    return (user_out, item_emb)


# speedup vs baseline: 3.5419x; 2.0612x over previous
import functools

import jax
import jax.numpy as jnp
from jax import lax
from jax.experimental import pallas as pl
from jax.experimental.pallas import tpu as pltpu
from jax.experimental.pallas import tpu_sc as plsc

USER_N = 5000
N = 10000
E = 320000
D = 128
L = 16
NC = 2
NS = 16
NW = NC * NS
C = 112
CHUNKS = 90
EPW = CHUNKS * C
E_PAD = NW * EPW
ROWS_PER_SUB = 624


_GATHER_DN = lax.GatherDimensionNumbers(
    offset_dims=(), collapsed_slice_dims=(0,), start_index_map=(0,))


def _splat(v16, t):
    idx = jnp.full((L, 1), t, jnp.int32)
    return lax.gather(v16, idx, _GATHER_DN, (1,),
                      mode=lax.GatherScatterMode.PROMISE_IN_BOUNDS)


def _sc_layer_body(ego_hbm, col_hbm, row_hbm, val_hbm, out_hbm,
                   cidx_all, vals_all, ridx_a, ridx_b, rows_a, rows_b, acc,
                   gsem_a, gsem_b, gsem_a2, gsem_b2, rsem_a, rsem_b):
    c = lax.axis_index("c")
    s = lax.axis_index("s")
    w = s * NC + c

    pltpu.sync_copy(col_hbm.at[pl.ds(w * EPW, EPW)], cidx_all)
    pltpu.sync_copy(val_hbm.at[pl.ds(w * EPW, EPW)], vals_all)

    zero = jnp.zeros((L,), jnp.float32)

    def _zero_rows(i, carry):
        for j in range(D // L):
            rows_a[i, pl.ds(j * L, L)] = zero
        return carry

    lax.fori_loop(0, C, _zero_rows, 0)
    for b in range(5):
        pltpu.sync_copy(rows_a.at[pl.ds(0, C)],
                        acc.at[pl.ds(s * ROWS_PER_SUB + b * C, C)])
    pltpu.sync_copy(rows_a.at[pl.ds(0, ROWS_PER_SUB - 5 * C)],
                    acc.at[pl.ds(s * ROWS_PER_SUB + 5 * C, ROWS_PER_SUB - 5 * C)])

    @pl.when(s == NS - 1)
    def _zero_tail():
        pltpu.sync_copy(rows_a.at[pl.ds(0, N - NS * ROWS_PER_SUB)],
                        acc.at[pl.ds(NS * ROWS_PER_SUB, N - NS * ROWS_PER_SUB)])

    plsc.subcore_barrier()

    base0 = w * EPW
    CS = 64

    def _gather(k, rows_t, gsem_t, gsem_t2):
        pltpu.async_copy(ego_hbm.at[cidx_all.at[pl.ds(k * C, CS)]],
                         rows_t.at[pl.ds(0, CS)], gsem_t)
        pltpu.async_copy(ego_hbm.at[cidx_all.at[pl.ds(k * C + CS, C - CS)]],
                         rows_t.at[pl.ds(CS, C - CS)], gsem_t2)

    def _gather_wait(k, rows_t, gsem_t, gsem_t2):
        pltpu.make_async_copy(ego_hbm.at[cidx_all.at[pl.ds(k * C, CS)]],
                              rows_t.at[pl.ds(0, CS)], gsem_t).wait()
        pltpu.make_async_copy(ego_hbm.at[cidx_all.at[pl.ds(k * C + CS, C - CS)]],
                              rows_t.at[pl.ds(CS, C - CS)], gsem_t2).wait()

    pltpu.async_copy(row_hbm.at[pl.ds(base0, C)], ridx_a, rsem_a)
    _gather(0, rows_a, gsem_a, gsem_a2)

    def _process(k, rows_x, ridx_x, gsem_x, gsem_x2, rsem_x,
                 rows_y, ridx_y, gsem_y, gsem_y2, rsem_y):
        @pl.when(k + 1 < CHUNKS)
        def _prefetch():
            pltpu.async_copy(row_hbm.at[pl.ds(base0 + (k + 1) * C, C)],
                             ridx_y, rsem_y)
            _gather(k + 1, rows_y, gsem_y, gsem_y2)

        _gather_wait(k, rows_x, gsem_x, gsem_x2)

        def _scale16(g, inner):
            v16 = vals_all[pl.ds(k * C + g * L, L)]
            for t in range(L):
                sp = _splat(v16, t)
                e = g * L + t
                for j in range(D // L):
                    rows_x[e, pl.ds(j * L, L)] = rows_x[e, pl.ds(j * L, L)] * sp
            return inner

        lax.fori_loop(0, C // L, _scale16, 0)
        pltpu.make_async_copy(row_hbm.at[pl.ds(base0 + k * C, C)],
                              ridx_x, rsem_x).wait()
        pltpu.sync_copy(rows_x, acc.at[ridx_x], add=True)

    def _chunk(k, carry):
        @pl.when(k % 2 == 0)
        def _even():
            _process(k, rows_a, ridx_a, gsem_a, gsem_a2, rsem_a,
                     rows_b, ridx_b, gsem_b, gsem_b2, rsem_b)

        @pl.when(k % 2 == 1)
        def _odd():
            _process(k, rows_b, ridx_b, gsem_b, gsem_b2, rsem_b,
                     rows_a, ridx_a, gsem_a, gsem_a2, rsem_a)

        return carry

    lax.fori_loop(0, CHUNKS, _chunk, 0)
    plsc.subcore_barrier()

    pltpu.sync_copy(acc.at[pl.ds(s * ROWS_PER_SUB, ROWS_PER_SUB)],
                    out_hbm.at[pl.ds(c * N + s * ROWS_PER_SUB, ROWS_PER_SUB)])

    @pl.when(s == NS - 1)
    def _write_tail():
        pltpu.sync_copy(acc.at[pl.ds(NS * ROWS_PER_SUB, N - NS * ROWS_PER_SUB)],
                        out_hbm.at[pl.ds(c * N + NS * ROWS_PER_SUB,
                                         N - NS * ROWS_PER_SUB)])


_sc_layer = functools.partial(
    pl.kernel,
    out_type=jax.ShapeDtypeStruct((NC * N, D), jnp.float32),
    mesh=plsc.VectorSubcoreMesh(core_axis_name="c", subcore_axis_name="s"),
    scratch_types=[
        pltpu.VMEM((EPW,), jnp.int32),
        pltpu.VMEM((EPW,), jnp.float32),
        pltpu.VMEM((C,), jnp.int32),
        pltpu.VMEM((C,), jnp.int32),
        pltpu.VMEM((C, D), jnp.float32),
        pltpu.VMEM((C, D), jnp.float32),
        pltpu.VMEM_SHARED((N, D), jnp.float32),
        pltpu.SemaphoreType.DMA,
        pltpu.SemaphoreType.DMA,
        pltpu.SemaphoreType.DMA,
        pltpu.SemaphoreType.DMA,
        pltpu.SemaphoreType.DMA,
        pltpu.SemaphoreType.DMA,
    ],
)(_sc_layer_body)


def _add_halves_body(a_ref, b_ref, o_ref):
    o_ref[...] = a_ref[...] + b_ref[...]


def _add_halves(p):
    return pl.pallas_call(
        _add_halves_body,
        out_shape=jax.ShapeDtypeStruct((N, D), jnp.float32),
        grid=(10,),
        in_specs=[pl.BlockSpec((N // 10, D), lambda i: (i, 0)),
                  pl.BlockSpec((N // 10, D), lambda i: (i + 10, 0))],
        out_specs=pl.BlockSpec((N // 10, D), lambda i: (i, 0)),
    )(p, p)


def _mean_body(p3a_ref, p3b_ref, e1_ref, e2_ref, o_ref):
    o_ref[...] = (p3a_ref[...] + p3b_ref[...] + e1_ref[...] + e2_ref[...]) * (
        jnp.float32(1.0 / 3.0))


def _user_mean(p3, e1, e2):
    blk = USER_N // 5
    return pl.pallas_call(
        _mean_body,
        out_shape=jax.ShapeDtypeStruct((USER_N, D), jnp.float32),
        grid=(5,),
        in_specs=[
            pl.BlockSpec((blk, D), lambda i: (i, 0)),
            pl.BlockSpec((blk, D), lambda i: (i + N // blk, 0)),
            pl.BlockSpec((blk, D), lambda i: (i, 0)),
            pl.BlockSpec((blk, D), lambda i: (i, 0)),
        ],
        out_specs=pl.BlockSpec((blk, D), lambda i: (i, 0)),
    )(p3, p3, e1, e2)


def kernel(user_emb, item_emb, adj_values, adj_indices):
    ego0 = jnp.concatenate([user_emb, item_emb], axis=0)
    row = adj_indices[0].astype(jnp.int32)
    col = adj_indices[1].astype(jnp.int32)
    pad = E_PAD - E
    zpad_i = jnp.zeros((pad,), jnp.int32)
    colp = jnp.concatenate([col, zpad_i])
    rowp = jnp.concatenate([row, zpad_i])
    valp = jnp.concatenate([adj_values, jnp.zeros((pad,), jnp.float32)])

    p1 = _sc_layer(ego0, colp, rowp, valp)
    e1 = _add_halves(p1)
    p2 = _sc_layer(e1, colp, rowp, valp)
    e2 = _add_halves(p2)
    p3 = _sc_layer(e2, colp, rowp, valp)
    user_out = _user_mean(p3, e1, e2)
    return (user_out, item_emb)
